# baseline (speedup 1.0000x reference)
import functools

import jax
import jax.numpy as jnp
from jax.experimental import pallas as pl
from jax.experimental.pallas import tpu as pltpu

N = 10000
D = 128
C = 16
K = 5000
R = 10
M = N // R


def _body(x_ref, w_ref, b_ref, wp_ref, xf_ref, pred_ref, ps_ref, zs_ref):
    j = pl.program_id(0)
    x = x_ref[:, :]
    w = w_ref[:, :]
    b = b_ref[:, :]
    wp = wp_ref[:, :]

    pred = jax.lax.dot_general(
        x, w, (((1,), (1,)), ((), ())), preferred_element_type=jnp.float32
    ) + b
    pred_ref[:, :] = pred
    ps_ref[pl.ds(j * M, M), :] = pred

    zj = jax.lax.dot_general(
        wp, pred, (((1,), (1,)), ((), ())),
        preferred_element_type=jnp.float32)
    zs_ref[pl.ds(j, 1), :] = zj

    @pl.when(j == R - 1)
    def _finale():
        z = zs_ref[:, :]

        u = jax.lax.bitcast_convert_type(z, jnp.uint32)
        sign = u >> jnp.uint32(31)
        flip = jnp.where(sign == jnp.uint32(1),
                         jnp.uint32(0xFFFFFFFF), jnp.uint32(0x80000000))
        key = u ^ flip

        def _cnt_ge(t):
            return jnp.sum((key >= t).astype(jnp.int32))

        kth = jnp.uint32(0)
        for sh in range(28, -1, -4):
            cnts = [_cnt_ge(kth | jnp.uint32(d << sh)) for d in range(1, 16)]
            digit = sum((c >= K).astype(jnp.uint32) for c in cnts)
            kth = kth | (digit << jnp.uint32(sh))

        above = key > kth
        m = jnp.sum(above.astype(jnp.int32))
        need = K - m

        tie = key == kth
        idx = (jax.lax.broadcasted_iota(jnp.int32, (R, M), 0) * M
               + jax.lax.broadcasted_iota(jnp.int32, (R, M), 1))

        def _cnt_lt(v):
            return jnp.sum((tie & (idx < v)).astype(jnp.int32))

        jstar = jnp.int32(0)
        for sh in range(12, -1, -4):
            cnts = [_cnt_lt(jstar | jnp.int32(d << sh)) for d in range(1, 16)]
            digit = sum((c < need).astype(jnp.int32) for c in cnts)
            jstar = jstar | (digit << sh)

        sel = above | (tie & (idx <= jstar))
        norm = jnp.sqrt(jnp.sum(wp * wp)) + 1e-16
        wgt = jnp.where(sel, jnp.tanh(z / norm), 0.0)

        acc = jnp.zeros((1, C), dtype=jnp.float32)
        for r in range(R):
            acc = acc + jax.lax.dot_general(
                wgt[r:r + 1, :], ps_ref[r * M:(r + 1) * M, :],
                (((1,), (0,)), ((), ())), preferred_element_type=jnp.float32)
        xf_ref[:, :] = acc * (1.0 / K)


@functools.partial(jax.jit, static_argnames=())
def kernel(x, edge_index, batch, W, b, w_pool):
    del edge_index, batch
    b2 = b.reshape(1, C)
    wp2 = w_pool.reshape(1, C)
    x_final, pred = pl.pallas_call(
        _body,
        grid=(R,),
        in_specs=[
            pl.BlockSpec((M, D), lambda j: (j, 0)),
            pl.BlockSpec((C, D), lambda j: (0, 0)),
            pl.BlockSpec((1, C), lambda j: (0, 0)),
            pl.BlockSpec((1, C), lambda j: (0, 0)),
        ],
        out_specs=(
            pl.BlockSpec((1, C), lambda j: (0, 0)),
            pl.BlockSpec((M, C), lambda j: (j, 0)),
        ),
        out_shape=(
            jax.ShapeDtypeStruct((1, C), jnp.float32),
            jax.ShapeDtypeStruct((N, C), jnp.float32),
        ),
        scratch_shapes=[
            pltpu.VMEM((N, C), jnp.float32),
            pltpu.VMEM((R, M), jnp.float32),
        ],
    )(x, W, b2, wp2)
    return (x_final, pred)


# speedup vs baseline: 1.9741x; 1.9741x over previous
import functools

import jax
import jax.numpy as jnp
from jax.experimental import pallas as pl
from jax.experimental.pallas import tpu as pltpu

N = 10000
D = 128
C = 16
K = 5000
R = 10
M = N // R


def _body(x_ref, w_ref, b_ref, wp_ref, xf_ref, predt_ref, zr_ref, zs_ref,
          ws_ref):
    x = x_ref[:, :]
    w = w_ref[:, :]
    bt = b_ref[:, :]
    wp = wp_ref[:, :]

    predt = jax.lax.dot_general(
        w, x, (((1,), (1,)), ((), ())), preferred_element_type=jnp.float32
    ) + bt
    predt_ref[:, :] = predt

    z = jax.lax.dot_general(
        wp, predt, (((1,), (0,)), ((), ())),
        preferred_element_type=jnp.float32)
    zr_ref[:, :] = z

    for j in range(R):
        zs_ref[j:j + 1, :] = zr_ref[0:1, pl.ds(j * M, M)]
    zd = zs_ref[:, :]

    u = jax.lax.bitcast_convert_type(zd, jnp.uint32)
    sign = u >> jnp.uint32(31)
    flip = jnp.where(sign == jnp.uint32(1),
                     jnp.uint32(0xFFFFFFFF), jnp.uint32(0x80000000))
    key = u ^ flip

    def _cnt_ge(t):
        return jnp.sum((key >= t).astype(jnp.int32))

    kth = jnp.uint32(0)
    for sh in range(28, -1, -4):
        cnts = [_cnt_ge(kth | jnp.uint32(d << sh)) for d in range(1, 16)]
        digit = sum((c >= K).astype(jnp.uint32) for c in cnts)
        kth = kth | (digit << jnp.uint32(sh))

    above = key > kth
    m = jnp.sum(above.astype(jnp.int32))
    need = K - m

    tie = key == kth
    idx = (jax.lax.broadcasted_iota(jnp.int32, (R, M), 0) * M
           + jax.lax.broadcasted_iota(jnp.int32, (R, M), 1))

    def _cnt_lt(v):
        return jnp.sum((tie & (idx < v)).astype(jnp.int32))

    jstar = jnp.int32(0)
    for sh in range(12, -1, -4):
        cnts = [_cnt_lt(jstar | jnp.int32(d << sh)) for d in range(1, 16)]
        digit = sum((c < need).astype(jnp.int32) for c in cnts)
        jstar = jstar | (digit << sh)

    sel = above | (tie & (idx <= jstar))
    norm = jnp.sqrt(jnp.sum(wp * wp)) + 1e-16
    wgt = jnp.where(sel, jnp.tanh(zd / norm), 0.0)

    for j in range(R):
        ws_ref[0:1, pl.ds(j * M, M)] = wgt[j:j + 1, :]

    acc = jax.lax.dot_general(
        ws_ref[:, :], predt, (((1,), (1,)), ((), ())),
        preferred_element_type=jnp.float32)
    xf_ref[:, :] = acc * (1.0 / K)


@functools.partial(jax.jit, static_argnames=())
def kernel(x, edge_index, batch, W, b, w_pool):
    del edge_index, batch
    bt = b.reshape(C, 1)
    wp2 = w_pool.reshape(1, C)
    x_final, predt = pl.pallas_call(
        _body,
        out_shape=(
            jax.ShapeDtypeStruct((1, C), jnp.float32),
            jax.ShapeDtypeStruct((C, N), jnp.float32),
        ),
        scratch_shapes=[
            pltpu.VMEM((1, N), jnp.float32),
            pltpu.VMEM((R, M), jnp.float32),
            pltpu.VMEM((1, N), jnp.float32),
        ],
    )(x, W, bt, wp2)
    return (x_final, predt.T)
